# baseline (speedup 1.0000x reference)
import functools

import jax
import jax.numpy as jnp
from jax import lax
from jax.experimental import pallas as pl
from jax.experimental.pallas import tpu as pltpu
from jax.experimental.pallas import tpu_sc as plsc

SL, BS, HS = 2048, 4, 1024
E, TOPK, FF = 64, 2, 1024
N_TOK = SL * BS
N_CPY = N_TOK * TOPK
T = 128
G = N_CPY // T + E
P = G * T

NW = 32
CPW = N_CPY // NW
TPW = N_TOK // NW
DCH = 32
NDC = CPW // DCH
CCH = 16
NCC = TPW // CCH


def _routing_body(e_ref, dest_ref, texp_ref, xblk_ref):
    CH = 512
    NCH = N_CPY // CH
    iota_e = lambda n: lax.broadcasted_iota(jnp.int32, (n, E), 1)

    def hist_step(c, hist):
        e_b = e_ref[pl.ds(c * CH, CH), :]
        onehot = (e_b == iota_e(CH)).astype(jnp.float32)
        return hist + jnp.sum(onehot, axis=0, keepdims=True)

    hist = lax.fori_loop(0, NCH, hist_step, jnp.zeros((1, E), jnp.float32))
    tiles = jnp.floor((hist + (T - 1)) * (1.0 / T))

    csum = tiles
    k = 1
    while k < E:
        csum = csum + jnp.concatenate(
            [jnp.zeros((1, k), jnp.float32), csum[:, : E - k]], axis=1)
        k *= 2
    tile_off = csum - tiles
    tile_end = csum
    row_off = tile_off * float(T)

    gg = lax.broadcasted_iota(jnp.int32, (G, E), 0).astype(jnp.float32)
    te = jnp.sum((jnp.broadcast_to(tile_end, (G, E)) <= gg).astype(jnp.int32),
                 axis=1, keepdims=True)
    total = jnp.sum(tiles, axis=1, keepdims=True)
    eids = lax.broadcasted_iota(jnp.int32, (1, E), 1)
    last_e = jnp.max(jnp.where(tiles > 0.0, eids, -1), axis=1, keepdims=True)
    gi = lax.broadcasted_iota(jnp.int32, (G, 1), 0)
    used = gi.astype(jnp.float32) < jnp.broadcast_to(total, (G, 1))
    texp_ref[...] = jnp.where(used, jnp.minimum(te, E - 1),
                              jnp.broadcast_to(last_e, (G, 1)))
    ui = jnp.broadcast_to(total, (G, 1)).astype(jnp.int32) - 1
    xblk_ref[...] = jnp.where(used, gi, ui)

    def dest_step(c, run_hist):
        e_b = e_ref[pl.ds(c * CH, CH), :]
        onehot = (e_b == iota_e(CH)).astype(jnp.float32)
        cs = onehot
        k = 1
        while k < CH:
            cs = cs + jnp.concatenate(
                [jnp.zeros((k, E), jnp.float32), cs[: CH - k, :]], axis=0)
            k *= 2
        base = jnp.broadcast_to(row_off + run_hist, (CH, E))
        d = jnp.sum(onehot * (base + cs - 1.0), axis=1, keepdims=True)
        dest_ref[pl.ds(c * CH, CH), :] = d.astype(jnp.int32)
        return run_hist + jnp.sum(onehot, axis=0, keepdims=True)

    lax.fori_loop(0, NCH, dest_step, jnp.zeros((1, E), jnp.float32))


def _routing(e_t):
    return pl.pallas_call(
        _routing_body,
        out_shape=[
            jax.ShapeDtypeStruct((N_CPY, 1), jnp.int32),
            jax.ShapeDtypeStruct((G, 1), jnp.int32),
            jax.ShapeDtypeStruct((G, 1), jnp.int32),
        ],
    )(e_t)


def _disperse_body(x_hbm, dest_hbm, w_hbm, grouped_hbm, wsort_hbm,
                   idx_v, xbuf, wbuf, sem1, sem2):
    wid = lax.axis_index("s") * 2 + lax.axis_index("c")
    jbase = wid * CPW
    rbase = (wid % (NW // TOPK)) * CPW

    descs = [None] * NDC
    for c in range(NDC):
        b = c & 1
        if c >= 2:
            descs[c - 2][0].wait()
            descs[c - 2][1].wait()
        off = c * DCH
        pltpu.sync_copy(dest_hbm.at[pl.ds(jbase + off, DCH)], idx_v[b])
        pltpu.sync_copy(x_hbm.at[pl.ds(rbase + off, DCH)], xbuf[b])
        pltpu.sync_copy(w_hbm.at[pl.ds(jbase + off, DCH)], wbuf[b])
        d1 = pltpu.async_copy(xbuf[b], grouped_hbm.at[idx_v[b]], sem1[b])
        d2 = pltpu.async_copy(wbuf[b], wsort_hbm.at[idx_v[b]], sem2[b])
        descs[c] = (d1, d2)
    for c in (NDC - 2, NDC - 1):
        descs[c][0].wait()
        descs[c][1].wait()


def _disperse(x_flat, dest_t, w_t):
    mesh = plsc.VectorSubcoreMesh(core_axis_name="c", subcore_axis_name="s")
    f = pl.kernel(
        _disperse_body,
        out_type=[
            jax.ShapeDtypeStruct((P, HS), jnp.float32),
            jax.ShapeDtypeStruct((P,), jnp.float32),
        ],
        mesh=mesh,
        scratch_types=[
            [pltpu.VMEM((DCH,), jnp.int32)] * 2,
            [pltpu.VMEM((DCH, HS), jnp.float32)] * 2,
            [pltpu.VMEM((DCH,), jnp.float32)] * 2,
            [pltpu.SemaphoreType.DMA] * 2,
            [pltpu.SemaphoreType.DMA] * 2,
        ],
    )
    return f(x_flat, dest_t, w_t)


def _gemm_body(texp_s, xblk_s, x_ref, w1_ref, w2_ref, ws_ref, out_ref,
               w1b_ref, w2b_ref):
    g = pl.program_id(0)
    active = xblk_s[g] == g

    @pl.when(active)
    def _():
        changed = jnp.logical_or(g == 0, texp_s[g] != texp_s[jnp.maximum(g, 1) - 1])

        @pl.when(changed)
        def _():
            w1b_ref[...] = w1_ref[0].astype(jnp.bfloat16)
            w2b_ref[...] = w2_ref[0].astype(jnp.bfloat16)

        h = jax.nn.gelu(
            jnp.dot(x_ref[...].astype(jnp.bfloat16), w1b_ref[...],
                    preferred_element_type=jnp.float32))
        out_ref[...] = jnp.dot(
            h.astype(jnp.bfloat16), w2b_ref[...],
            preferred_element_type=jnp.float32) * ws_ref[...]


def _gemm(texp, xblk, grouped, w1, w2, wsort, interpret=False):
    grid_spec = pltpu.PrefetchScalarGridSpec(
        num_scalar_prefetch=2,
        grid=(G,),
        in_specs=[
            pl.BlockSpec((T, HS), lambda g, t, b: (b[g], 0)),
            pl.BlockSpec((1, HS, FF), lambda g, t, b: (t[g], 0, 0)),
            pl.BlockSpec((1, FF, HS), lambda g, t, b: (t[g], 0, 0)),
            pl.BlockSpec((T, 1), lambda g, t, b: (b[g], 0)),
        ],
        out_specs=pl.BlockSpec((T, HS), lambda g, t, b: (b[g], 0)),
        scratch_shapes=[
            pltpu.VMEM((HS, FF), jnp.bfloat16),
            pltpu.VMEM((FF, HS), jnp.bfloat16),
        ],
    )
    return pl.pallas_call(
        _gemm_body,
        grid_spec=grid_spec,
        out_shape=jax.ShapeDtypeStruct((P, HS), jnp.float32),
        interpret=interpret,
    )(texp, xblk, grouped, w1, w2, wsort)


def _combine_body(dest_hbm, mid_hbm, out_hbm, idx_a, idx_b, buf_a, buf_b,
                  sem_a, sem_b, sem_o):
    wid = lax.axis_index("s") * 2 + lax.axis_index("c")
    tbase = wid * TPW

    def fetch(c, b):
        off = tbase + c * CCH
        pltpu.sync_copy(dest_hbm.at[pl.ds(off, CCH)], idx_a[b])
        pltpu.sync_copy(dest_hbm.at[pl.ds(N_TOK + off, CCH)], idx_b[b])
        da = pltpu.async_copy(mid_hbm.at[idx_a[b]], buf_a[b], sem_a[b])
        db = pltpu.async_copy(mid_hbm.at[idx_b[b]], buf_b[b], sem_b[b])
        return da, db

    def process(c, b, da, db):
        da.wait()
        db.wait()

        def row_add(r, _):
            for v in range(HS // 16):
                sl = pl.ds(v * 16, 16)
                buf_a[b][r, sl] = buf_a[b][r, sl] + buf_b[b][r, sl]
            return 0

        lax.fori_loop(0, CCH, row_add, 0)
        off = tbase + c * CCH
        return pltpu.async_copy(buf_a[b], out_hbm.at[pl.ds(off, CCH)], sem_o[b])

    descs = [None] * NCC
    outd = [None] * NCC
    for c in range(NCC):
        b = c & 1
        if c >= 2:
            outd[c - 2].wait()
        descs[c] = fetch(c, b)
        if c >= 1:
            outd[c - 1] = process(c - 1, (c - 1) & 1, *descs[c - 1])
    outd[NCC - 1] = process(NCC - 1, (NCC - 1) & 1, *descs[NCC - 1])
    outd[NCC - 2].wait()
    outd[NCC - 1].wait()


def _combine(dest_t, mid):
    mesh = plsc.VectorSubcoreMesh(core_axis_name="c", subcore_axis_name="s")
    f = pl.kernel(
        _combine_body,
        out_type=jax.ShapeDtypeStruct((N_TOK, HS), jnp.float32),
        mesh=mesh,
        scratch_types=[
            [pltpu.VMEM((CCH,), jnp.int32)] * 2,
            [pltpu.VMEM((CCH,), jnp.int32)] * 2,
            [pltpu.VMEM((CCH, HS), jnp.float32)] * 2,
            [pltpu.VMEM((CCH, HS), jnp.float32)] * 2,
            [pltpu.SemaphoreType.DMA] * 2,
            [pltpu.SemaphoreType.DMA] * 2,
            [pltpu.SemaphoreType.DMA] * 2,
        ],
    )
    return f(dest_t, mid)


def kernel(x, expert_weights, expert_indices, w1, w2):
    x_flat = x.reshape(N_TOK, HS)
    e_t = expert_indices.T.reshape(N_CPY, 1).astype(jnp.int32)
    w_t = expert_weights.T.reshape(N_CPY)

    dest, texp, xblk = _routing(e_t)
    grouped, wsort = _disperse(x_flat, dest.reshape(N_CPY), w_t)
    mid = _gemm(texp.reshape(G), xblk.reshape(G), grouped, w1, w2,
                wsort.reshape(P, 1))
    out = _combine(dest.reshape(N_CPY), mid)
    return out.reshape(x.shape)


# speedup vs baseline: 1.0363x; 1.0363x over previous
import functools

import jax
import jax.numpy as jnp
from jax import lax
from jax.experimental import pallas as pl
from jax.experimental.pallas import tpu as pltpu
from jax.experimental.pallas import tpu_sc as plsc

SL, BS, HS = 2048, 4, 1024
E, TOPK, FF = 64, 2, 1024
N_TOK = SL * BS
N_CPY = N_TOK * TOPK
T = 128
G = N_CPY // T + E
P = G * T

NW = 32
CPW = N_CPY // NW
TPW = N_TOK // NW
DCH = 32
NDC = TPW // DCH
CCH = 16
NCC = TPW // CCH


def _routing_body(e_ref, dest_ref, texp_ref, xblk_ref):
    CH = 512
    NCH = N_CPY // CH
    iota_e = lambda n: lax.broadcasted_iota(jnp.int32, (n, E), 1)

    def hist_step(c, hist):
        e_b = e_ref[pl.ds(c * CH, CH), :]
        onehot = (e_b == iota_e(CH)).astype(jnp.float32)
        return hist + jnp.sum(onehot, axis=0, keepdims=True)

    hist = lax.fori_loop(0, NCH, hist_step, jnp.zeros((1, E), jnp.float32))
    tiles = jnp.floor((hist + (T - 1)) * (1.0 / T))

    csum = tiles
    k = 1
    while k < E:
        csum = csum + jnp.concatenate(
            [jnp.zeros((1, k), jnp.float32), csum[:, : E - k]], axis=1)
        k *= 2
    tile_off = csum - tiles
    tile_end = csum
    row_off = tile_off * float(T)

    gg = lax.broadcasted_iota(jnp.int32, (G, E), 0).astype(jnp.float32)
    te = jnp.sum((jnp.broadcast_to(tile_end, (G, E)) <= gg).astype(jnp.int32),
                 axis=1, keepdims=True)
    total = jnp.sum(tiles, axis=1, keepdims=True)
    eids = lax.broadcasted_iota(jnp.int32, (1, E), 1)
    last_e = jnp.max(jnp.where(tiles > 0.0, eids, -1), axis=1, keepdims=True)
    gi = lax.broadcasted_iota(jnp.int32, (G, 1), 0)
    used = gi.astype(jnp.float32) < jnp.broadcast_to(total, (G, 1))
    texp_ref[...] = jnp.where(used, jnp.minimum(te, E - 1),
                              jnp.broadcast_to(last_e, (G, 1)))
    ui = jnp.broadcast_to(total, (G, 1)).astype(jnp.int32) - 1
    xblk_ref[...] = jnp.where(used, gi, ui)

    def dest_step(c, run_hist):
        e_b = e_ref[pl.ds(c * CH, CH), :]
        onehot = (e_b == iota_e(CH)).astype(jnp.float32)
        cs = onehot
        k = 1
        while k < CH:
            cs = cs + jnp.concatenate(
                [jnp.zeros((k, E), jnp.float32), cs[: CH - k, :]], axis=0)
            k *= 2
        base = jnp.broadcast_to(row_off + run_hist, (CH, E))
        d = jnp.sum(onehot * (base + cs - 1.0), axis=1, keepdims=True)
        dest_ref[pl.ds(c * CH, CH), :] = d.astype(jnp.int32)
        return run_hist + jnp.sum(onehot, axis=0, keepdims=True)

    lax.fori_loop(0, NCH, dest_step, jnp.zeros((1, E), jnp.float32))


def _routing(e_t):
    return pl.pallas_call(
        _routing_body,
        out_shape=[
            jax.ShapeDtypeStruct((N_CPY, 1), jnp.int32),
            jax.ShapeDtypeStruct((G, 1), jnp.int32),
            jax.ShapeDtypeStruct((G, 1), jnp.int32),
        ],
    )(e_t)


def _disperse_body(x_hbm, dest2_hbm, w2_hbm, grouped_hbm, wsort_hbm,
                   idx2, wv2, xbuf, semx, sems, semw):
    wid = lax.axis_index("s") * 2 + lax.axis_index("c")
    rbase = wid * TPW
    crow = wid * (TPW // DCH)

    nb = N_TOK // DCH
    pltpu.sync_copy(dest2_hbm.at[pl.ds(crow, NDC)], idx2.at[pl.ds(0, NDC)])
    pltpu.sync_copy(dest2_hbm.at[pl.ds(nb + crow, NDC)],
                    idx2.at[pl.ds(NDC, NDC)])
    pltpu.sync_copy(w2_hbm.at[pl.ds(crow, NDC)], wv2.at[pl.ds(0, NDC)])
    pltpu.sync_copy(w2_hbm.at[pl.ds(nb + crow, NDC)], wv2.at[pl.ds(NDC, NDC)])

    ld = [None] * NDC
    st = [None] * NDC

    def scatter(c):
        b = c & 1
        ld[c].wait()
        s1 = pltpu.async_copy(xbuf[b], grouped_hbm.at[idx2.at[c]], sems[b])
        s2 = pltpu.async_copy(xbuf[b], grouped_hbm.at[idx2.at[NDC + c]],
                              sems[b])
        w1_ = pltpu.async_copy(wv2.at[c], wsort_hbm.at[idx2.at[c]], semw[b])
        w2_ = pltpu.async_copy(wv2.at[NDC + c], wsort_hbm.at[idx2.at[NDC + c]],
                               semw[b])
        return (s1, s2, w1_, w2_)

    for c in range(NDC):
        b = c & 1
        if c >= 2:
            for d in st[c - 2]:
                d.wait()
        ld[c] = pltpu.async_copy(x_hbm.at[pl.ds(rbase + c * DCH, DCH)],
                                 xbuf[b], semx[b])
        if c >= 1:
            st[c - 1] = scatter(c - 1)
    st[NDC - 1] = scatter(NDC - 1)
    for c in (NDC - 2, NDC - 1):
        for d in st[c]:
            d.wait()


def _disperse(x_flat, dest2, w2):
    mesh = plsc.VectorSubcoreMesh(core_axis_name="c", subcore_axis_name="s")
    f = pl.kernel(
        _disperse_body,
        out_type=[
            jax.ShapeDtypeStruct((P, HS), jnp.float32),
            jax.ShapeDtypeStruct((P,), jnp.float32),
        ],
        mesh=mesh,
        scratch_types=[
            pltpu.VMEM((2 * NDC, DCH), jnp.int32),
            pltpu.VMEM((2 * NDC, DCH), jnp.float32),
            [pltpu.VMEM((DCH, HS), jnp.float32)] * 2,
            [pltpu.SemaphoreType.DMA] * 2,
            [pltpu.SemaphoreType.DMA] * 2,
            [pltpu.SemaphoreType.DMA] * 2,
        ],
    )
    return f(x_flat, dest2, w2)


def _gemm_body(texp_s, xblk_s, x_ref, w1_ref, w2_ref, ws_ref, out_ref,
               w1b_ref, w2b_ref):
    g = pl.program_id(0)
    active = xblk_s[g] == g

    @pl.when(active)
    def _():
        changed = jnp.logical_or(g == 0, texp_s[g] != texp_s[jnp.maximum(g, 1) - 1])

        @pl.when(changed)
        def _():
            w1b_ref[...] = w1_ref[0].astype(jnp.bfloat16)
            w2b_ref[...] = w2_ref[0].astype(jnp.bfloat16)

        h = jax.nn.gelu(
            jnp.dot(x_ref[...].astype(jnp.bfloat16), w1b_ref[...],
                    preferred_element_type=jnp.float32))
        out_ref[...] = jnp.dot(
            h.astype(jnp.bfloat16), w2b_ref[...],
            preferred_element_type=jnp.float32) * ws_ref[...]


def _gemm(texp, xblk, grouped, w1, w2, wsort, interpret=False):
    grid_spec = pltpu.PrefetchScalarGridSpec(
        num_scalar_prefetch=2,
        grid=(G,),
        in_specs=[
            pl.BlockSpec((T, HS), lambda g, t, b: (b[g], 0)),
            pl.BlockSpec((1, HS, FF), lambda g, t, b: (t[g], 0, 0)),
            pl.BlockSpec((1, FF, HS), lambda g, t, b: (t[g], 0, 0)),
            pl.BlockSpec((T, 1), lambda g, t, b: (b[g], 0)),
        ],
        out_specs=pl.BlockSpec((T, HS), lambda g, t, b: (b[g], 0)),
        scratch_shapes=[
            pltpu.VMEM((HS, FF), jnp.bfloat16),
            pltpu.VMEM((FF, HS), jnp.bfloat16),
        ],
    )
    return pl.pallas_call(
        _gemm_body,
        grid_spec=grid_spec,
        out_shape=jax.ShapeDtypeStruct((P, HS), jnp.float32),
        interpret=interpret,
    )(texp, xblk, grouped, w1, w2, wsort)


def _combine_body(dest2_hbm, mid_hbm, out_hbm, idx2, buf_a, buf_b,
                  sem_a, sem_b, sem_o):
    wid = lax.axis_index("s") * 2 + lax.axis_index("c")
    tbase = wid * TPW
    crow = wid * NCC
    nb = N_TOK // CCH

    pltpu.sync_copy(dest2_hbm.at[pl.ds(crow, NCC)], idx2.at[pl.ds(0, NCC)])
    pltpu.sync_copy(dest2_hbm.at[pl.ds(nb + crow, NCC)],
                    idx2.at[pl.ds(NCC, NCC)])

    def fetch(c, b):
        da = pltpu.async_copy(mid_hbm.at[idx2.at[c]], buf_a[b], sem_a[b])
        db = pltpu.async_copy(mid_hbm.at[idx2.at[NCC + c]], buf_b[b], sem_b[b])
        return da, db

    def process(c, b, da, db):
        da.wait()
        db.wait()

        def row_add(r, _):
            for v in range(HS // 16):
                sl = pl.ds(v * 16, 16)
                buf_a[b][r, sl] = buf_a[b][r, sl] + buf_b[b][r, sl]
            return 0

        lax.fori_loop(0, CCH, row_add, 0)
        off = tbase + c * CCH
        return pltpu.async_copy(buf_a[b], out_hbm.at[pl.ds(off, CCH)], sem_o[b])

    descs = [None] * NCC
    outd = [None] * NCC
    for c in range(NCC):
        b = c & 1
        if c >= 2:
            outd[c - 2].wait()
        descs[c] = fetch(c, b)
        if c >= 1:
            outd[c - 1] = process(c - 1, (c - 1) & 1, *descs[c - 1])
    outd[NCC - 1] = process(NCC - 1, (NCC - 1) & 1, *descs[NCC - 1])
    outd[NCC - 2].wait()
    outd[NCC - 1].wait()


def _combine(dest2c, mid):
    mesh = plsc.VectorSubcoreMesh(core_axis_name="c", subcore_axis_name="s")
    f = pl.kernel(
        _combine_body,
        out_type=jax.ShapeDtypeStruct((N_TOK, HS), jnp.float32),
        mesh=mesh,
        scratch_types=[
            pltpu.VMEM((2 * NCC, CCH), jnp.int32),
            [pltpu.VMEM((CCH, HS), jnp.float32)] * 2,
            [pltpu.VMEM((CCH, HS), jnp.float32)] * 2,
            [pltpu.SemaphoreType.DMA] * 2,
            [pltpu.SemaphoreType.DMA] * 2,
            [pltpu.SemaphoreType.DMA] * 2,
        ],
    )
    return f(dest2c, mid)


def kernel(x, expert_weights, expert_indices, w1, w2):
    x_flat = x.reshape(N_TOK, HS)
    e_t = expert_indices.T.reshape(N_CPY, 1).astype(jnp.int32)
    w_t = expert_weights.T.reshape(N_CPY)

    dest, texp, xblk = _routing(e_t)
    grouped, wsort = _disperse(x_flat, dest.reshape(N_CPY // DCH, DCH),
                               w_t.reshape(N_CPY // DCH, DCH))
    mid = _gemm(texp.reshape(G), xblk.reshape(G), grouped, w1, w2,
                wsort.reshape(P, 1))
    out = _combine(dest.reshape(N_CPY // CCH, CCH), mid)
    return out.reshape(x.shape)
